# baseline (speedup 1.0000x reference)
import functools

import jax
import jax.numpy as jnp
from jax import lax
from jax.experimental import pallas as pl
from jax.experimental.pallas import tpu as pltpu
from jax.experimental.pallas import tpu_sc as plsc

_START = 126
_STOP = 127
_LN2 = 0.6931471805599453
_SHIFT = 6.0


def _gold_sc_body(feat_hbm, trans_hbm, len_hbm, tags_hbm, out_hbm,
                  tags_v, len_v, fidx, tidx, fval, tval, obuf, sem):
    c = lax.axis_index("c")
    s = lax.axis_index("s")
    wid = s * 2 + c
    b = wid // 2
    h = wid % 2
    cp_t = pltpu.async_copy(tags_hbm.at[b], tags_v, sem)
    cp_l = pltpu.async_copy(len_hbm, len_v.at[pl.ds(0, 16)], sem)
    cp_t.wait()
    cp_l.wait()

    i32 = jnp.int32
    iota = lax.iota(i32, 16)
    bvec = jnp.zeros((16,), i32) + b
    shift_idx = jnp.maximum(iota - 1, 0)
    lenb = len_v[pl.ds(0, 16)].at[bvec].get(mode="promise_in_bounds")
    base = h * 256
    prev = tags_v[pl.ds(240, 16)]
    prev_last = prev.at[jnp.zeros((16,), i32) + 15].get(
        mode="promise_in_bounds")
    fbase = (b * 512 + base) * 128
    for i in range(16):
        tg = tags_v[pl.ds(base + i * 16, 16)]
        fidx[pl.ds(i * 16, 16)] = fbase + (iota + i * 16) * 128 + tg
    cps = [pltpu.async_copy(feat_hbm.at[fidx.at[pl.ds(j * 128, 128)]],
                            fval.at[pl.ds(j * 128, 128)], sem)
           for j in range(2)]
    for i in range(16):
        t = iota + (i * 16 + base)
        tg = tags_v[pl.ds(base + i * 16, 16)]
        tgm1 = jnp.where(
            iota == 0, prev_last,
            tg.at[shift_idx].get(mode="promise_in_bounds"))
        prev_last = tg.at[jnp.zeros((16,), i32) + 15].get(
            mode="promise_in_bounds")
        start = jnp.where(t == 0, _START, tgm1)
        stop = jnp.where(t >= lenb, _STOP, tg)
        tidx[pl.ds(i * 16, 16)] = stop * 128 + start
    cps += [pltpu.async_copy(trans_hbm.at[tidx.at[pl.ds(j * 128, 128)]],
                             tval.at[pl.ds(j * 128, 128)], sem)
            for j in range(2)]
    for cp in cps:
        cp.wait()
    acc = jnp.zeros((16,), jnp.float32)
    for i in range(16):
        t = iota + (i * 16 + base)
        acc = acc + jnp.where(t < lenb, fval[pl.ds(i * 16, 16)], 0.0)
        acc = acc + jnp.where(t <= lenb, tval[pl.ds(i * 16, 16)], 0.0)
    obuf[...] = acc
    pltpu.sync_copy(obuf, out_hbm.at[wid])


def _gold_sc(features, transitions, lengths, tags):
    B, T, K = features.shape
    run = pl.kernel(
        _gold_sc_body,
        mesh=plsc.VectorSubcoreMesh(core_axis_name="c", subcore_axis_name="s"),
        out_type=jax.ShapeDtypeStruct((32, 16), jnp.float32),
        scratch_types=[
            pltpu.VMEM((512,), jnp.int32),
            pltpu.VMEM((128,), jnp.int32),
            pltpu.VMEM((256,), jnp.int32),
            pltpu.VMEM((256,), jnp.int32),
            pltpu.VMEM((256,), jnp.float32),
            pltpu.VMEM((256,), jnp.float32),
            pltpu.VMEM((16,), jnp.float32),
            pltpu.SemaphoreType.DMA,
        ],
    )
    return run(features.reshape(B * T * K), transitions.reshape(K * K),
               lengths, tags)


_UNROLL = 8
_SPLIT = 32


def _scan_half(first):
    def body(f_ref, trans_ref, len_ref, *refs):
        B, T, K = f_ref.shape
        f32 = jnp.float32
        exp_t = jnp.exp(trans_ref[...]).T.astype(jnp.bfloat16)
        lens = len_ref[...]
        UNROLL = _UNROLL

        def step(blk, carry):
            u, eint, rcap, ecap = carry
            tb = blk * UNROLL
            fchunk = f_ref[:, pl.ds(pl.multiple_of(tb, UNROLL), UNROLL), :]
            echunk = jnp.exp(fchunk - _SHIFT)
            for k in range(UNROLL):
                t = tb + k
                v = jax.lax.dot_general(
                    u.astype(jnp.bfloat16), exp_t, (((1,), (0,)), ((), ())),
                    preferred_element_type=f32)
                hit = lens == t
                rcap = jnp.where(hit, v[:, _STOP:_STOP + 1], rcap)
                ecap = jnp.where(hit, eint, ecap)
                u = echunk[:, k, :] * v
            m = jnp.max(u, axis=1, keepdims=True)
            ebits = lax.bitcast_convert_type(m, jnp.int32) & 0x7F800000
            inv = lax.bitcast_convert_type(0x7F000000 - ebits, f32)
            u = u * inv
            eint = eint + (ebits >> 23) - 127
            return u, eint, rcap, ecap

        nblocks = jnp.max(lens) // UNROLL + 1
        if first:
            (uo_ref, eo_ref, ro_ref, co_ref) = refs
            iota_b = jax.lax.broadcasted_iota(jnp.int32, (B, K), 1)
            u0 = (iota_b == _START).astype(f32)
            zi = jnp.zeros((B, 1), jnp.int32)
            carry0 = (u0, zi, jnp.ones((B, 1), f32), zi)
            u, eint, rcap, ecap = jax.lax.fori_loop(
                0, jnp.minimum(nblocks, _SPLIT), step, carry0)
            uo_ref[...], eo_ref[...] = u, eint
            ro_ref[...], co_ref[...] = rcap, ecap
        else:
            (ui_ref, ei_ref, ri_ref, ci_ref, out_ref) = refs
            carry0 = (ui_ref[...], ei_ref[...], ri_ref[...], ci_ref[...])
            _, _, rcap, ecap = jax.lax.fori_loop(
                _SPLIT, jnp.maximum(nblocks, _SPLIT), step, carry0)
            fwd = (jnp.log(rcap) + ecap.astype(f32) * _LN2
                   + _SHIFT * lens.astype(f32))
            out_ref[...] = jnp.reshape(jnp.sum(fwd), (1, 1))
    return body


def kernel(features, transitions, lengths, tags):
    B, T, K = features.shape
    tags = tags.astype(jnp.int32)
    lengths = lengths.astype(jnp.int32)

    f32 = jnp.float32
    len2 = lengths.reshape(B, 1)
    u, eint, rcap, ecap = pl.pallas_call(
        _scan_half(True),
        out_shape=(jax.ShapeDtypeStruct((B, K), f32),
                   jax.ShapeDtypeStruct((B, 1), jnp.int32),
                   jax.ShapeDtypeStruct((B, 1), f32),
                   jax.ShapeDtypeStruct((B, 1), jnp.int32)),
    )(features, transitions, len2)
    partials = _gold_sc(features, transitions, lengths, tags)
    fwd_sum = pl.pallas_call(
        _scan_half(False),
        out_shape=jax.ShapeDtypeStruct((1, 1), f32),
    )(features, transitions, len2, u, eint, rcap, ecap)
    return (fwd_sum.reshape(()) - jnp.sum(partials)) / B


# speedup vs baseline: 1.0357x; 1.0357x over previous
import functools

import jax
import jax.numpy as jnp
from jax import lax
from jax.experimental import pallas as pl
from jax.experimental.pallas import tpu as pltpu
from jax.experimental.pallas import tpu_sc as plsc

_START = 126
_STOP = 127
_LN2 = 0.6931471805599453
_SHIFT = 6.0


def _gold_sc_body(feat_hbm, trans_hbm, len_hbm, tags_hbm, out_hbm,
                  tags_v, len_v, fidx, tidx, fval, tval, obuf, sem):
    c = lax.axis_index("c")
    s = lax.axis_index("s")
    wid = s * 2 + c
    b = wid // 2
    h = wid % 2
    cp_t = pltpu.async_copy(tags_hbm.at[b], tags_v, sem)
    cp_l = pltpu.async_copy(len_hbm, len_v.at[pl.ds(0, 16)], sem)
    cp_t.wait()
    cp_l.wait()

    i32 = jnp.int32
    iota = lax.iota(i32, 16)
    bvec = jnp.zeros((16,), i32) + b
    shift_idx = jnp.maximum(iota - 1, 0)
    lenb = len_v[pl.ds(0, 16)].at[bvec].get(mode="promise_in_bounds")
    base = h * 256
    prev = tags_v[pl.ds(240, 16)]
    prev_last = prev.at[jnp.zeros((16,), i32) + 15].get(
        mode="promise_in_bounds")
    fbase = (b * 512 + base) * 128
    for i in range(16):
        tg = tags_v[pl.ds(base + i * 16, 16)]
        fidx[pl.ds(i * 16, 16)] = fbase + (iota + i * 16) * 128 + tg
    cps = [pltpu.async_copy(feat_hbm.at[fidx.at[pl.ds(j * 128, 128)]],
                            fval.at[pl.ds(j * 128, 128)], sem)
           for j in range(2)]
    for i in range(16):
        t = iota + (i * 16 + base)
        tg = tags_v[pl.ds(base + i * 16, 16)]
        tgm1 = jnp.where(
            iota == 0, prev_last,
            tg.at[shift_idx].get(mode="promise_in_bounds"))
        prev_last = tg.at[jnp.zeros((16,), i32) + 15].get(
            mode="promise_in_bounds")
        start = jnp.where(t == 0, _START, tgm1)
        stop = jnp.where(t >= lenb, _STOP, tg)
        tidx[pl.ds(i * 16, 16)] = stop * 128 + start
    cps += [pltpu.async_copy(trans_hbm.at[tidx.at[pl.ds(j * 128, 128)]],
                             tval.at[pl.ds(j * 128, 128)], sem)
            for j in range(2)]
    for cp in cps:
        cp.wait()
    acc = jnp.zeros((16,), jnp.float32)
    for i in range(16):
        t = iota + (i * 16 + base)
        acc = acc + jnp.where(t < lenb, fval[pl.ds(i * 16, 16)], 0.0)
        acc = acc + jnp.where(t <= lenb, tval[pl.ds(i * 16, 16)], 0.0)
    obuf[...] = acc
    pltpu.sync_copy(obuf, out_hbm.at[wid])


def _gold_sc(features, transitions, lengths, tags):
    B, T, K = features.shape
    run = pl.kernel(
        _gold_sc_body,
        mesh=plsc.VectorSubcoreMesh(core_axis_name="c", subcore_axis_name="s"),
        out_type=jax.ShapeDtypeStruct((32, 16), jnp.float32),
        scratch_types=[
            pltpu.VMEM((512,), jnp.int32),
            pltpu.VMEM((128,), jnp.int32),
            pltpu.VMEM((256,), jnp.int32),
            pltpu.VMEM((256,), jnp.int32),
            pltpu.VMEM((256,), jnp.float32),
            pltpu.VMEM((256,), jnp.float32),
            pltpu.VMEM((16,), jnp.float32),
            pltpu.SemaphoreType.DMA,
        ],
    )
    return run(features.reshape(B * T * K), transitions.reshape(K * K),
               lengths, tags)


def _fwd_kernel(f_ref, trans_ref, len_ref, out_ref):
    B, T, K = f_ref.shape
    f32 = jnp.float32

    exp_t = jnp.exp(trans_ref[...]).T.astype(jnp.bfloat16)

    iota_b = jax.lax.broadcasted_iota(jnp.int32, (B, K), 1)
    u0 = (iota_b == _START).astype(f32)
    lens = len_ref[...]

    UNROLL = 8

    def step(blk, carry):
        u, eint, rcap, ecap = carry
        tb = blk * UNROLL
        fchunk = f_ref[:, pl.ds(pl.multiple_of(tb, UNROLL), UNROLL), :]
        echunk = jnp.exp(fchunk - _SHIFT)
        for k in range(UNROLL):
            t = tb + k
            v = jax.lax.dot_general(
                u.astype(jnp.bfloat16), exp_t, (((1,), (0,)), ((), ())),
                preferred_element_type=f32)
            hit = lens == t
            rcap = jnp.where(hit, v[:, _STOP:_STOP + 1], rcap)
            ecap = jnp.where(hit, eint, ecap)
            u = echunk[:, k, :] * v
        m = jnp.max(u, axis=1, keepdims=True)
        ebits = lax.bitcast_convert_type(m, jnp.int32) & 0x7F800000
        inv = lax.bitcast_convert_type(0x7F000000 - ebits, f32)
        u = u * inv
        eint = eint + (ebits >> 23) - 127
        return u, eint, rcap, ecap

    zi = jnp.zeros((B, 1), jnp.int32)
    nblocks = jnp.max(lens) // UNROLL + 1
    _, _, rcap, ecap = jax.lax.fori_loop(
        0, nblocks, step, (u0, zi, jnp.ones((B, 1), f32), zi))

    fwd = (jnp.log(rcap) + ecap.astype(f32) * _LN2
           + _SHIFT * lens.astype(f32))
    out_ref[...] = jnp.reshape(jnp.sum(fwd), (1, 1))


def kernel(features, transitions, lengths, tags):
    B, T, K = features.shape
    tags = tags.astype(jnp.int32)
    lengths = lengths.astype(jnp.int32)

    fwd_sum = pl.pallas_call(
        _fwd_kernel,
        out_shape=jax.ShapeDtypeStruct((1, 1), jnp.float32),
    )(features, transitions, lengths.reshape(B, 1))
    partials = _gold_sc(features, transitions, lengths, tags)
    return (fwd_sum.reshape(()) - jnp.sum(partials)) / B
